# Initial kernel scaffold; baseline (speedup 1.0000x reference)
import jax
import jax.numpy as jnp
from jax.experimental import pallas as pl


def kernel(tertiary, subgraph_indices):
    raise NotImplementedError("write your pallas kernel here")


# speedup vs baseline: 11.1665x; 11.1665x over previous
import dataclasses
import functools

import jax
import jax.numpy as jnp
from jax import lax
from jax.experimental import pallas as pl
from jax.experimental.pallas import tpu as pltpu
from jax.experimental.pallas import tpu_sc as plsc

_N = 4096
_NEAR = 10
_FAR = 5
_K = 16
_LOW, _HIGH = 0.0, 20.0
_BLK = 256

_NC = 2
_NS = 16
_L = 16
_ROWS_PER_TILE = _N // (_NC * _NS)


def _rne_bf16(x):
    u = lax.bitcast_convert_type(x, jnp.uint32)
    r = (u + jnp.uint32(0x7FFF) + ((u >> 16) & jnp.uint32(1)))
    return lax.bitcast_convert_type(r & jnp.uint32(0xFFFF0000), jnp.float32)


def _tc_body(rows_ref, cols_ref, conn_ref):
    xi = rows_ref[:, 0:1]
    yi = rows_ref[:, 1:2]
    zi = rows_ref[:, 2:3]
    sqi = rows_ref[:, 3:4]
    xs = cols_ref[0:1, :]
    ys = cols_ref[1:2, :]
    zs = cols_ref[2:3, :]
    sqs = cols_ref[3:4, :]

    dot = xi * xs + yi * ys + zi * zs
    d2 = (sqi + sqs) - 2.0 * dot
    dist = jnp.sqrt(jnp.maximum(d2, 1e-12))

    iota = lax.broadcasted_iota(jnp.int32, (_BLK, _N), 1)
    inf = jnp.float32(jnp.inf)

    work = dist
    for k in range(_NEAR):
        m = jnp.min(work, axis=1, keepdims=True)
        idx = jnp.min(jnp.where(work == m, iota, _N), axis=1, keepdims=True)
        work = jnp.where(iota == idx, inf, work)
        conn_ref[:, k:k + 1] = idx
    work = dist
    for k in range(_FAR):
        m = jnp.max(work, axis=1, keepdims=True)
        idx = jnp.min(jnp.where(work == m, iota, _N), axis=1, keepdims=True)
        work = jnp.where(iota == idx, -inf, work)
        conn_ref[:, _NEAR + k:_NEAR + k + 1] = idx
    conn_ref[:, 15:16] = jnp.zeros((_BLK, 1), jnp.int32)


def _select_conn(pos, interpret=False):
    sq = jnp.sum(pos * pos, axis=-1, keepdims=True)
    pb = _rne_bf16(pos)
    rows = jnp.concatenate(
        [pb, sq, jnp.zeros((_N, 4), jnp.float32)], axis=1)
    cols = rows.T
    return pl.pallas_call(
        _tc_body,
        grid=(_N // _BLK,),
        in_specs=[
            pl.BlockSpec((_BLK, 8), lambda i: (i, 0)),
            pl.BlockSpec((8, _N), lambda i: (0, 0)),
        ],
        out_specs=pl.BlockSpec((_BLK, 16), lambda i: (i, 0)),
        out_shape=jax.ShapeDtypeStruct((_N, 16), jnp.int32),
        interpret=interpret,
    )(rows, cols)


def _newton_sqrt(s):
    u = plsc.bitcast(s, jnp.uint32)
    y = plsc.bitcast((u >> 1) + jnp.uint32(0x1FBD1DF5), jnp.float32)
    for _ in range(4):
        y = 0.5 * (y + s / y)
    return y


def _sc_featurize(px, py, pz, conn):
    gamma = (_HIGH - _LOW) / (_K - 1)
    inv = jnp.float32(1.0 / (2.0 * gamma * gamma))

    cp = pltpu.CompilerParams()
    if "needs_layout_passes" in pltpu.CompilerParams.__dataclass_fields__:
        cp = dataclasses.replace(cp, needs_layout_passes=False)

    out_per_tile = _ROWS_PER_TILE * _K * _L

    @functools.partial(
        pl.kernel,
        out_type=jax.ShapeDtypeStruct((_N * _K * _L,), jnp.float32),
        mesh=plsc.VectorSubcoreMesh(core_axis_name="c", subcore_axis_name="s"),
        compiler_params=cp,
        scratch_types=[
            pltpu.VMEM((_N,), jnp.float32),
            pltpu.VMEM((_N,), jnp.float32),
            pltpu.VMEM((_N,), jnp.float32),
            pltpu.VMEM((_ROWS_PER_TILE * 16,), jnp.int32),
            pltpu.VMEM((out_per_tile,), jnp.float32),
        ],
    )
    def sc_kernel(px_hbm, py_hbm, pz_hbm, conn_hbm, out_hbm,
                  px_v, py_v, pz_v, conn_v, out_v):
        wid = lax.axis_index("s") * _NC + lax.axis_index("c")
        base = wid * _ROWS_PER_TILE
        pltpu.sync_copy(px_hbm, px_v)
        pltpu.sync_copy(py_hbm, py_v)
        pltpu.sync_copy(pz_hbm, pz_v)
        pltpu.sync_copy(conn_hbm.at[pl.ds(base * 16, _ROWS_PER_TILE * 16)],
                        conn_v)

        @pl.loop(0, _ROWS_PER_TILE)
        def _(r):
            gidx = base + r
            self_idx = jnp.full((_L,), gidx, jnp.int32)
            cj = conn_v[pl.ds(r * 16, _L)]
            xj = plsc.load_gather(px_v, [cj])
            yj = plsc.load_gather(py_v, [cj])
            zj = plsc.load_gather(pz_v, [cj])
            xi = plsc.load_gather(px_v, [self_idx])
            yi = plsc.load_gather(py_v, [self_idx])
            zi = plsc.load_gather(pz_v, [self_idx])
            dx = (xi - xj) + 1e-6
            dy = (yi - yj) + 1e-6
            dz = (zi - zj) + 1e-6
            s = dx * dx + dy * dy + dz * dz
            d = _newton_sqrt(s)
            for k in range(_K):
                diff = d - jnp.float32(k * gamma)
                out_v[pl.ds((r * _K + k) * _L, _L)] = (
                    jnp.exp(-(diff * diff) * inv))

        pltpu.sync_copy(out_v, out_hbm.at[pl.ds(base * _K * _L,
                                                out_per_tile)])

    out = sc_kernel(px, py, pz, conn.reshape(-1))
    return out.reshape(_N, _K, _L).transpose(0, 2, 1)[:, :_NEAR + _FAR, :]


@jax.jit
def _run(pos):
    conn = _select_conn(pos)
    return _sc_featurize(pos[:, 0], pos[:, 1], pos[:, 2], conn)


def kernel(tertiary, subgraph_indices):
    pos = tertiary[:, 1]
    return _run(pos)


# speedup vs baseline: 11.1736x; 1.0006x over previous
import dataclasses
import functools

import jax
import jax.numpy as jnp
from jax import lax
from jax.experimental import pallas as pl
from jax.experimental.pallas import tpu as pltpu
from jax.experimental.pallas import tpu_sc as plsc

_N = 4096
_NEAR = 10
_FAR = 5
_K = 16
_LOW, _HIGH = 0.0, 20.0
_BLK = 256

_NC = 2
_NS = 16
_L = 16
_ROWS_PER_TILE = _N // (_NC * _NS)


def _rne_bf16(x):
    u = lax.bitcast_convert_type(x, jnp.uint32)
    r = (u + jnp.uint32(0x7FFF) + ((u >> 16) & jnp.uint32(1)))
    return lax.bitcast_convert_type(r & jnp.uint32(0xFFFF0000), jnp.float32)


def _tc_body(rows_ref, cols_ref, conn_ref):
    xi = rows_ref[:, 0:1]
    yi = rows_ref[:, 1:2]
    zi = rows_ref[:, 2:3]
    sqi = rows_ref[:, 3:4]
    xs = cols_ref[0:1, :]
    ys = cols_ref[1:2, :]
    zs = cols_ref[2:3, :]
    sqs = cols_ref[3:4, :]

    dot = xi * xs + yi * ys + zi * zs
    d2 = (sqi + sqs) - 2.0 * dot
    dist = jnp.sqrt(jnp.maximum(d2, 1e-12))

    iota = lax.broadcasted_iota(jnp.int32, (_BLK, _N), 1)
    inf = jnp.float32(jnp.inf)

    work = dist
    for k in range(_NEAR):
        m = jnp.min(work, axis=1, keepdims=True)
        idx = jnp.min(jnp.where(work == m, iota, _N), axis=1, keepdims=True)
        work = jnp.where(iota == idx, inf, work)
        conn_ref[:, k:k + 1] = idx
    work = dist
    for k in range(_FAR):
        m = jnp.max(work, axis=1, keepdims=True)
        idx = jnp.min(jnp.where(work == m, iota, _N), axis=1, keepdims=True)
        work = jnp.where(iota == idx, -inf, work)
        conn_ref[:, _NEAR + k:_NEAR + k + 1] = idx
    conn_ref[:, 15:16] = jnp.zeros((_BLK, 1), jnp.int32)


def _select_conn(pos, interpret=False):
    sq = jnp.sum(pos * pos, axis=-1, keepdims=True)
    pb = _rne_bf16(pos)
    rows = jnp.concatenate(
        [pb, sq, jnp.zeros((_N, 4), jnp.float32)], axis=1)
    cols = rows.T
    return pl.pallas_call(
        _tc_body,
        grid=(_N // _BLK,),
        in_specs=[
            pl.BlockSpec((_BLK, 8), lambda i: (i, 0)),
            pl.BlockSpec((8, _N), lambda i: (0, 0)),
        ],
        out_specs=pl.BlockSpec((_BLK, 16), lambda i: (i, 0)),
        out_shape=jax.ShapeDtypeStruct((_N, 16), jnp.int32),
        compiler_params=pltpu.CompilerParams(
            dimension_semantics=("parallel",)),
        interpret=interpret,
    )(rows, cols)


def _newton_sqrt(s):
    u = plsc.bitcast(s, jnp.uint32)
    y = plsc.bitcast((u >> 1) + jnp.uint32(0x1FBD1DF5), jnp.float32)
    for _ in range(4):
        y = 0.5 * (y + s / y)
    return y


def _sc_featurize(px, py, pz, conn):
    gamma = (_HIGH - _LOW) / (_K - 1)
    inv = jnp.float32(1.0 / (2.0 * gamma * gamma))

    cp = pltpu.CompilerParams()
    if "needs_layout_passes" in pltpu.CompilerParams.__dataclass_fields__:
        cp = dataclasses.replace(cp, needs_layout_passes=False)

    out_per_tile = _ROWS_PER_TILE * _K * _L

    @functools.partial(
        pl.kernel,
        out_type=jax.ShapeDtypeStruct((_N * _K * _L,), jnp.float32),
        mesh=plsc.VectorSubcoreMesh(core_axis_name="c", subcore_axis_name="s"),
        compiler_params=cp,
        scratch_types=[
            pltpu.VMEM((_N,), jnp.float32),
            pltpu.VMEM((_N,), jnp.float32),
            pltpu.VMEM((_N,), jnp.float32),
            pltpu.VMEM((_ROWS_PER_TILE * 16,), jnp.int32),
            pltpu.VMEM((out_per_tile,), jnp.float32),
        ],
    )
    def sc_kernel(px_hbm, py_hbm, pz_hbm, conn_hbm, out_hbm,
                  px_v, py_v, pz_v, conn_v, out_v):
        wid = lax.axis_index("s") * _NC + lax.axis_index("c")
        base = wid * _ROWS_PER_TILE
        pltpu.sync_copy(px_hbm, px_v)
        pltpu.sync_copy(py_hbm, py_v)
        pltpu.sync_copy(pz_hbm, pz_v)
        pltpu.sync_copy(conn_hbm.at[pl.ds(base * 16, _ROWS_PER_TILE * 16)],
                        conn_v)

        @pl.loop(0, _ROWS_PER_TILE)
        def _(r):
            gidx = base + r
            self_idx = jnp.full((_L,), gidx, jnp.int32)
            cj = conn_v[pl.ds(r * 16, _L)]
            xj = plsc.load_gather(px_v, [cj])
            yj = plsc.load_gather(py_v, [cj])
            zj = plsc.load_gather(pz_v, [cj])
            xi = plsc.load_gather(px_v, [self_idx])
            yi = plsc.load_gather(py_v, [self_idx])
            zi = plsc.load_gather(pz_v, [self_idx])
            dx = (xi - xj) + 1e-6
            dy = (yi - yj) + 1e-6
            dz = (zi - zj) + 1e-6
            s = dx * dx + dy * dy + dz * dz
            d = _newton_sqrt(s)
            for k in range(_K):
                diff = d - jnp.float32(k * gamma)
                out_v[pl.ds((r * _K + k) * _L, _L)] = (
                    jnp.exp(-(diff * diff) * inv))

        pltpu.sync_copy(out_v, out_hbm.at[pl.ds(base * _K * _L,
                                                out_per_tile)])

    out = sc_kernel(px, py, pz, conn.reshape(-1))
    return out.reshape(_N, _K, _L).transpose(0, 2, 1)[:, :_NEAR + _FAR, :]


@jax.jit
def _run(pos):
    conn = _select_conn(pos)
    return _sc_featurize(pos[:, 0], pos[:, 1], pos[:, 2], conn)


def kernel(tertiary, subgraph_indices):
    pos = tertiary[:, 1]
    return _run(pos)


# speedup vs baseline: 14.7742x; 1.3222x over previous
import dataclasses
import functools

import jax
import jax.numpy as jnp
from jax import lax
from jax.experimental import pallas as pl
from jax.experimental.pallas import tpu as pltpu
from jax.experimental.pallas import tpu_sc as plsc

_N = 4096
_NEAR = 10
_FAR = 5
_K = 16
_LOW, _HIGH = 0.0, 20.0
_BLK = 256

_NC = 2
_NS = 16
_L = 16
_ROWS_PER_TILE = _N // (_NC * _NS)


def _rne_bf16(x):
    u = lax.bitcast_convert_type(x, jnp.uint32)
    r = (u + jnp.uint32(0x7FFF) + ((u >> 16) & jnp.uint32(1)))
    return lax.bitcast_convert_type(r & jnp.uint32(0xFFFF0000), jnp.float32)


def _tc_body(rows_ref, cols_ref, conn_ref):
    xi = rows_ref[:, 0:1]
    yi = rows_ref[:, 1:2]
    zi = rows_ref[:, 2:3]
    sqi = rows_ref[:, 3:4]
    xs = cols_ref[0:1, :]
    ys = cols_ref[1:2, :]
    zs = cols_ref[2:3, :]
    sqs = cols_ref[3:4, :]

    dot = xi * xs + yi * ys + zi * zs
    d2 = (sqi + sqs) - 2.0 * dot
    dist = jnp.sqrt(jnp.maximum(d2, 1e-12))

    db = lax.bitcast_convert_type(dist, jnp.int32) & jnp.int32(~0xFFF)
    iota = lax.broadcasted_iota(jnp.int32, (_BLK, _N), 1)

    work = db | iota
    for k in range(_NEAR):
        m = jnp.min(work, axis=1, keepdims=True)
        work = jnp.where(work == m, jnp.int32(0x7FFFFFFF), work)
        conn_ref[:, k:k + 1] = m & jnp.int32(0xFFF)
    work = db | (jnp.int32(0xFFF) - iota)
    for k in range(_FAR):
        m = jnp.max(work, axis=1, keepdims=True)
        work = jnp.where(work == m, jnp.int32(0), work)
        conn_ref[:, _NEAR + k:_NEAR + k + 1] = (
            jnp.int32(0xFFF) - (m & jnp.int32(0xFFF)))
    conn_ref[:, 15:16] = jnp.zeros((_BLK, 1), jnp.int32)


def _select_conn(pos, interpret=False):
    sq = jnp.sum(pos * pos, axis=-1, keepdims=True)
    pb = _rne_bf16(pos)
    rows = jnp.concatenate(
        [pb, sq, jnp.zeros((_N, 4), jnp.float32)], axis=1)
    cols = rows.T
    return pl.pallas_call(
        _tc_body,
        grid=(_N // _BLK,),
        in_specs=[
            pl.BlockSpec((_BLK, 8), lambda i: (i, 0)),
            pl.BlockSpec((8, _N), lambda i: (0, 0)),
        ],
        out_specs=pl.BlockSpec((_BLK, 16), lambda i: (i, 0)),
        out_shape=jax.ShapeDtypeStruct((_N, 16), jnp.int32),
        compiler_params=pltpu.CompilerParams(
            dimension_semantics=("parallel",)),
        interpret=interpret,
    )(rows, cols)


def _newton_sqrt(s):
    u = plsc.bitcast(s, jnp.uint32)
    y = plsc.bitcast((u >> 1) + jnp.uint32(0x1FBD1DF5), jnp.float32)
    for _ in range(4):
        y = 0.5 * (y + s / y)
    return y


def _sc_featurize(px, py, pz, conn):
    gamma = (_HIGH - _LOW) / (_K - 1)
    inv = jnp.float32(1.0 / (2.0 * gamma * gamma))

    cp = pltpu.CompilerParams()
    if "needs_layout_passes" in pltpu.CompilerParams.__dataclass_fields__:
        cp = dataclasses.replace(cp, needs_layout_passes=False)

    out_per_tile = _ROWS_PER_TILE * _K * _L

    @functools.partial(
        pl.kernel,
        out_type=jax.ShapeDtypeStruct((_N * _K * _L,), jnp.float32),
        mesh=plsc.VectorSubcoreMesh(core_axis_name="c", subcore_axis_name="s"),
        compiler_params=cp,
        scratch_types=[
            pltpu.VMEM((_N,), jnp.float32),
            pltpu.VMEM((_N,), jnp.float32),
            pltpu.VMEM((_N,), jnp.float32),
            pltpu.VMEM((_ROWS_PER_TILE * 16,), jnp.int32),
            pltpu.VMEM((out_per_tile,), jnp.float32),
        ],
    )
    def sc_kernel(px_hbm, py_hbm, pz_hbm, conn_hbm, out_hbm,
                  px_v, py_v, pz_v, conn_v, out_v):
        wid = lax.axis_index("s") * _NC + lax.axis_index("c")
        base = wid * _ROWS_PER_TILE
        pltpu.sync_copy(px_hbm, px_v)
        pltpu.sync_copy(py_hbm, py_v)
        pltpu.sync_copy(pz_hbm, pz_v)
        pltpu.sync_copy(conn_hbm.at[pl.ds(base * 16, _ROWS_PER_TILE * 16)],
                        conn_v)

        @pl.loop(0, _ROWS_PER_TILE)
        def _(r):
            gidx = base + r
            self_idx = jnp.full((_L,), gidx, jnp.int32)
            cj = conn_v[pl.ds(r * 16, _L)]
            xj = plsc.load_gather(px_v, [cj])
            yj = plsc.load_gather(py_v, [cj])
            zj = plsc.load_gather(pz_v, [cj])
            xi = plsc.load_gather(px_v, [self_idx])
            yi = plsc.load_gather(py_v, [self_idx])
            zi = plsc.load_gather(pz_v, [self_idx])
            dx = (xi - xj) + 1e-6
            dy = (yi - yj) + 1e-6
            dz = (zi - zj) + 1e-6
            s = dx * dx + dy * dy + dz * dz
            d = _newton_sqrt(s)
            for k in range(_K):
                diff = d - jnp.float32(k * gamma)
                out_v[pl.ds((r * _K + k) * _L, _L)] = (
                    jnp.exp(-(diff * diff) * inv))

        pltpu.sync_copy(out_v, out_hbm.at[pl.ds(base * _K * _L,
                                                out_per_tile)])

    out = sc_kernel(px, py, pz, conn.reshape(-1))
    return out.reshape(_N, _K, _L).transpose(0, 2, 1)[:, :_NEAR + _FAR, :]


@jax.jit
def _run(pos):
    conn = _select_conn(pos)
    return _sc_featurize(pos[:, 0], pos[:, 1], pos[:, 2], conn)


def kernel(tertiary, subgraph_indices):
    pos = tertiary[:, 1]
    return _run(pos)
